# speedup vs baseline: 8.5292x; 8.5292x over previous
import jax
import jax.numpy as jnp
from jax.experimental import pallas as pl
from jax.experimental.pallas import tpu as pltpu

_OUT_D = 8192
_IN_D = 2048
_B = 128
_K = 164
_BOOST = 100.0
_JBLK = 1024
_NJ = _OUT_D // _JBLK


def _pooler_kernel(x_ref, conn_ref, avg_ref, out_ref):
    j = pl.program_id(0)
    avg = avg_ref[...]
    s = jnp.sum(avg)
    avg_blk = avg_ref[:, pl.ds(j * _JBLK, _JBLK)]
    neigh = (s - avg_blk) / (_OUT_D - 1)
    boost = jnp.exp(-_BOOST * (avg_blk - neigh))
    ov = jnp.dot(x_ref[...], conn_ref[...], preferred_element_type=jnp.float32)
    out_ref[:, pl.ds(j * _JBLK, _JBLK)] = ov * boost

    @pl.when(j == _NJ - 1)
    def _select():
        vals = out_ref[...]
        u = jax.lax.bitcast_convert_type(vals, jnp.int32)

        def vbody(_, carry):
            lo, hi = carry
            mid = lo + jax.lax.div(hi - lo, 2)
            cnt = jnp.sum((u >= mid).astype(jnp.int32), axis=1, keepdims=True)
            ge = cnt >= _K
            return jnp.where(ge, mid, lo), jnp.where(ge, hi, mid)

        lo0 = jnp.zeros((_B, 1), jnp.int32)
        hi0 = jnp.full((_B, 1), jnp.int32(2**31 - 1))
        t, _ = jax.lax.fori_loop(0, 31, vbody, (lo0, hi0))

        gt = u > t
        c = jnp.sum(gt.astype(jnp.int32), axis=1, keepdims=True)
        m = _K - c
        eq = u == t
        idx = jax.lax.broadcasted_iota(jnp.int32, (_B, _OUT_D), 1)

        def ibody(_, carry):
            lo, hi = carry
            mid = lo + jax.lax.div(hi - lo, 2)
            cnt = jnp.sum((eq & (idx < mid)).astype(jnp.int32), axis=1,
                          keepdims=True)
            ge = cnt >= m
            return jnp.where(ge, lo, mid), jnp.where(ge, mid, hi)

        li0 = jnp.zeros((_B, 1), jnp.int32)
        hi0i = jnp.full((_B, 1), jnp.int32(_OUT_D))
        _, cut = jax.lax.fori_loop(0, 13, ibody, (li0, hi0i))

        mask = gt | (eq & (idx < cut))
        out_ref[...] = mask.astype(jnp.float32)


def kernel(x, connection, avg_activation):
    return pl.pallas_call(
        _pooler_kernel,
        grid=(_NJ,),
        in_specs=[
            pl.BlockSpec((_B, _IN_D), lambda j: (0, 0)),
            pl.BlockSpec((_IN_D, _JBLK), lambda j: (0, j)),
            pl.BlockSpec((1, _OUT_D), lambda j: (0, 0)),
        ],
        out_specs=pl.BlockSpec((_B, _OUT_D), lambda j: (0, 0)),
        out_shape=jax.ShapeDtypeStruct((_B, _OUT_D), jnp.float32),
    )(x, connection, avg_activation)


# speedup vs baseline: 9.3420x; 1.0953x over previous
import jax
import jax.numpy as jnp
from jax.experimental import pallas as pl
from jax.experimental.pallas import tpu as pltpu

_OUT_D = 8192
_IN_D = 2048
_B = 128
_K = 164
_BOOST = 100.0
_JBLK = 1024
_NJ = _OUT_D // _JBLK


def _count_ge(u, thr):
    return jnp.sum((u >= thr).astype(jnp.int32), axis=1, keepdims=True)


def _pooler_kernel(x_ref, conn_ref, avg_ref, out_ref, lo_ref, max_ref):
    j = pl.program_id(0)
    avg = avg_ref[...]
    s = jnp.sum(avg)
    avg_blk = avg_ref[:, pl.ds(j * _JBLK, _JBLK)]
    neigh = (s - avg_blk) / (_OUT_D - 1)
    boost = jnp.exp(-_BOOST * (avg_blk - neigh))
    ov = jnp.dot(x_ref[...], conn_ref[...], preferred_element_type=jnp.float32)
    ovb = ov * boost
    out_ref[:, pl.ds(j * _JBLK, _JBLK)] = ovb

    ub = jax.lax.bitcast_convert_type(ovb, jnp.int32)
    bmax = jnp.max(ub, axis=1, keepdims=True)

    @pl.when(j == 0)
    def _seed():
        max_ref[...] = bmax

        def vbody(_, carry):
            lo, hi = carry
            mid = lo + jax.lax.div(hi - lo, 2)
            ge = _count_ge(ub, mid) >= _K
            return jnp.where(ge, mid, lo), jnp.where(ge, hi, mid)

        lo0 = jnp.zeros((_B, 1), jnp.int32)
        hi0 = bmax + 1
        t0, _ = jax.lax.fori_loop(0, 31, vbody, (lo0, hi0))
        lo_ref[...] = t0

    @pl.when(j > 0)
    def _accum_max():
        max_ref[...] = jnp.maximum(max_ref[...], bmax)

    @pl.when(j == _NJ - 1)
    def _select():
        u = jax.lax.bitcast_convert_type(out_ref[...], jnp.int32)

        def vcond(carry):
            lo, hi = carry
            return jnp.any(hi - lo > 1)

        def vbody(carry):
            lo, hi = carry
            mid = lo + jax.lax.div(hi - lo, 2)
            ge = _count_ge(u, mid) >= _K
            return jnp.where(ge, mid, lo), jnp.where(ge, hi, mid)

        t, _ = jax.lax.while_loop(vcond, vbody, (lo_ref[...], max_ref[...] + 1))

        gt = u > t
        c = jnp.sum(gt.astype(jnp.int32), axis=1, keepdims=True)
        m = _K - c
        eq = u == t
        idx = jax.lax.broadcasted_iota(jnp.int32, (_B, _OUT_D), 1)

        def tcond(carry):
            need, _ = carry
            return jnp.max(need) > 0

        def tbody(carry):
            need, last = carry
            avail = eq & (idx > last)
            fi = jnp.min(jnp.where(avail, idx, _OUT_D), axis=1, keepdims=True)
            act = need > 0
            return need - act.astype(jnp.int32), jnp.where(act, fi, last)

        _, last = jax.lax.while_loop(
            tcond, tbody, (m, jnp.full((_B, 1), -1, jnp.int32)))

        out_ref[...] = (gt | (eq & (idx <= last))).astype(jnp.float32)


def kernel(x, connection, avg_activation):
    return pl.pallas_call(
        _pooler_kernel,
        grid=(_NJ,),
        in_specs=[
            pl.BlockSpec((_B, _IN_D), lambda j: (0, 0)),
            pl.BlockSpec((_IN_D, _JBLK), lambda j: (0, j)),
            pl.BlockSpec((1, _OUT_D), lambda j: (0, 0)),
        ],
        out_specs=pl.BlockSpec((_B, _OUT_D), lambda j: (0, 0)),
        out_shape=jax.ShapeDtypeStruct((_B, _OUT_D), jnp.float32),
        scratch_shapes=[
            pltpu.VMEM((_B, 1), jnp.int32),
            pltpu.VMEM((_B, 1), jnp.int32),
        ],
    )(x, connection, avg_activation)


# speedup vs baseline: 9.3681x; 1.0028x over previous
import jax
import jax.numpy as jnp
from jax.experimental import pallas as pl
from jax.experimental.pallas import tpu as pltpu

_OUT_D = 8192
_IN_D = 2048
_B = 128
_K = 164
_BOOST = 100.0
_JBLK = 1024
_NJ = _OUT_D // _JBLK


def _count_ge(u, thr):
    return jnp.sum((u >= thr).astype(jnp.int32), axis=1, keepdims=True)


def _pooler_kernel(x_ref, conn_ref, avg_ref, out_ref, lo_ref, max_ref):
    j = pl.program_id(0)
    avg = avg_ref[...]
    s = jnp.sum(avg)
    avg_blk = avg_ref[:, pl.ds(j * _JBLK, _JBLK)]
    neigh = (s - avg_blk) / (_OUT_D - 1)
    boost = jnp.exp(-_BOOST * (avg_blk - neigh))
    conn_b = conn_ref[...].astype(jnp.bfloat16)
    ov = jnp.dot(x_ref[...], conn_b, preferred_element_type=jnp.float32)
    ovb = ov * boost
    out_ref[:, pl.ds(j * _JBLK, _JBLK)] = ovb

    ub = jax.lax.bitcast_convert_type(ovb, jnp.int32)
    bmax = jnp.max(ub, axis=1, keepdims=True)

    @pl.when(j == 0)
    def _seed():
        max_ref[...] = bmax

        def vbody(_, carry):
            lo, hi = carry
            mid = lo + jax.lax.div(hi - lo, 2)
            ge = _count_ge(ub, mid) >= _K
            return jnp.where(ge, mid, lo), jnp.where(ge, hi, mid)

        lo0 = jnp.zeros((_B, 1), jnp.int32)
        hi0 = bmax + 1
        t0, _ = jax.lax.fori_loop(0, 31, vbody, (lo0, hi0))
        lo_ref[...] = t0

    @pl.when(j > 0)
    def _accum_max():
        max_ref[...] = jnp.maximum(max_ref[...], bmax)

    @pl.when(j == _NJ - 1)
    def _select():
        u = jax.lax.bitcast_convert_type(out_ref[...], jnp.int32)

        def vcond(carry):
            lo, hi = carry
            return jnp.any(hi - lo > 1)

        def vbody(carry):
            lo, hi = carry
            mid = lo + jax.lax.div(hi - lo, 2)
            ge = _count_ge(u, mid) >= _K
            return jnp.where(ge, mid, lo), jnp.where(ge, hi, mid)

        t, _ = jax.lax.while_loop(vcond, vbody, (lo_ref[...], max_ref[...] + 1))

        gt = u > t
        c = jnp.sum(gt.astype(jnp.int32), axis=1, keepdims=True)
        m = _K - c
        eq = u == t
        idx = jax.lax.broadcasted_iota(jnp.int32, (_B, _OUT_D), 1)

        def tcond(carry):
            need, _ = carry
            return jnp.max(need) > 0

        def tbody(carry):
            need, last = carry
            avail = eq & (idx > last)
            fi = jnp.min(jnp.where(avail, idx, _OUT_D), axis=1, keepdims=True)
            act = need > 0
            return need - act.astype(jnp.int32), jnp.where(act, fi, last)

        _, last = jax.lax.while_loop(
            tcond, tbody, (m, jnp.full((_B, 1), -1, jnp.int32)))

        out_ref[...] = (gt | (eq & (idx <= last))).astype(jnp.float32)


def kernel(x, connection, avg_activation):
    return pl.pallas_call(
        _pooler_kernel,
        grid=(_NJ,),
        in_specs=[
            pl.BlockSpec((_B, _IN_D), lambda j: (0, 0)),
            pl.BlockSpec((_IN_D, _JBLK), lambda j: (0, j)),
            pl.BlockSpec((1, _OUT_D), lambda j: (0, 0)),
        ],
        out_specs=pl.BlockSpec((_B, _OUT_D), lambda j: (0, 0)),
        out_shape=jax.ShapeDtypeStruct((_B, _OUT_D), jnp.float32),
        scratch_shapes=[
            pltpu.VMEM((_B, 1), jnp.int32),
            pltpu.VMEM((_B, 1), jnp.int32),
        ],
    )(x, connection, avg_activation)
